# baseline (speedup 1.0000x reference)
import functools

import jax
import jax.numpy as jnp
from jax import lax
from jax.experimental import pallas as pl
from jax.experimental.pallas import tpu as pltpu
from jax.experimental.pallas import tpu_sc as plsc

_H = 14
_W = 14
_HW = _H * _W


def _sc_gather(idx, table2):
    B = idx.shape[0]
    D2 = table2.shape[1]
    info = plsc.get_sparse_core_info()
    nw = info.num_cores * info.num_subcores
    b_per_w = B // nw
    mesh = plsc.VectorSubcoreMesh(core_axis_name="c", subcore_axis_name="s")

    @functools.partial(
        pl.kernel,
        mesh=mesh,
        out_type=jax.ShapeDtypeStruct((B, D2), jnp.float32),
        scratch_types=[
            pltpu.VMEM((b_per_w,), jnp.int32),
            pltpu.VMEM((b_per_w, D2), jnp.float32),
            pltpu.SemaphoreType.DMA,
        ],
    )
    def k(idx_hbm, table_hbm, out_hbm, idx_v, rows_v, sem):
        wid = lax.axis_index("s") * info.num_cores + lax.axis_index("c")
        base = wid * b_per_w
        pltpu.sync_copy(idx_hbm.at[pl.ds(base, b_per_w)], idx_v)
        pltpu.async_copy(table_hbm.at[idx_v], rows_v, sem).wait()
        pltpu.sync_copy(rows_v, out_hbm.at[pl.ds(base, b_per_w)])

    return k(idx, table2)


def _tc_select_broadcast(wide, parity, D):
    B = wide.shape[0]
    bb = 128
    DHW = D * _HW
    LANES = 128
    n_tiles = DHW // LANES

    def body(w_ref, p_ref, o_ref):
        w = w_ref[...]
        par = p_ref[...] > 0
        sel = jnp.where(par, w[:, D:], w[:, :D])
        lane = jax.lax.broadcasted_iota(jnp.int32, (bb, LANES), 1)
        for t in range(n_tiles):
            g0 = LANES * t
            d0 = g0 // _HW
            d1 = (g0 + LANES - 1) // _HW
            a = jnp.broadcast_to(sel[:, d0:d0 + 1], (bb, LANES))
            if d0 != d1:
                b2 = jnp.broadcast_to(sel[:, d1:d1 + 1], (bb, LANES))
                a = jnp.where(lane < (d0 + 1) * _HW - g0, a, b2)
            o_ref[:, g0:g0 + LANES] = a

    return pl.pallas_call(
        body,
        grid=(B // bb,),
        in_specs=[
            pl.BlockSpec((bb, 2 * D), lambda i: (i, 0)),
            pl.BlockSpec((bb, 1), lambda i: (i, 0)),
        ],
        out_specs=pl.BlockSpec((bb, DHW), lambda i: (i, 0)),
        out_shape=jax.ShapeDtypeStruct((B, DHW), jnp.float32),
    )(wide, parity)


def kernel(labels, table):
    B = labels.shape[0]
    V, D = table.shape
    labels = labels.astype(jnp.int32)
    table2 = table.reshape(V // 2, 2 * D)
    wide = _sc_gather(labels >> 1, table2)
    parity = (labels & 1).reshape(B, 1)
    out = _tc_select_broadcast(wide, parity, D)
    return out.reshape(B, D, _H, _W)


# speedup vs baseline: 4.5076x; 4.5076x over previous
import functools

import jax
import jax.numpy as jnp
from jax import lax
from jax.experimental import pallas as pl
from jax.experimental.pallas import tpu as pltpu
from jax.experimental.pallas import tpu_sc as plsc

_H = 14
_W = 14
_HW = _H * _W


def _sc_gather(idx, table2):
    B = idx.shape[0]
    D2 = table2.shape[1]
    info = plsc.get_sparse_core_info()
    nw = info.num_cores * info.num_subcores
    b_per_w = B // nw
    mesh = plsc.VectorSubcoreMesh(core_axis_name="c", subcore_axis_name="s")

    @functools.partial(
        pl.kernel,
        mesh=mesh,
        out_type=jax.ShapeDtypeStruct((B, D2), jnp.float32),
        scratch_types=[
            pltpu.VMEM((b_per_w,), jnp.int32),
            pltpu.VMEM((b_per_w, D2), jnp.float32),
            pltpu.SemaphoreType.DMA,
        ],
    )
    def k(idx_hbm, table_hbm, out_hbm, idx_v, rows_v, sem):
        wid = lax.axis_index("s") * info.num_cores + lax.axis_index("c")
        base = wid * b_per_w
        pltpu.sync_copy(idx_hbm.at[pl.ds(base, b_per_w)], idx_v)
        pltpu.async_copy(table_hbm.at[idx_v], rows_v, sem).wait()
        pltpu.sync_copy(rows_v, out_hbm.at[pl.ds(base, b_per_w)])

    return k(idx, table2)


def _tc_select_broadcast(wide, parity, D):
    B = wide.shape[0]
    k = 4

    def body(w_ref, p_ref, o_ref, e_ref):
        @pl.when(pl.program_id(0) == 0)
        def _():
            wt = jnp.transpose(w_ref[...])
            par = p_ref[...] > 0
            e_ref[...] = jnp.where(par, wt[D:], wt[:D])

        o_ref[...] = jnp.broadcast_to(e_ref[...][None], (k, D, B))

    return pl.pallas_call(
        body,
        grid=(_HW // k,),
        in_specs=[
            pl.BlockSpec((B, 2 * D), lambda i: (0, 0)),
            pl.BlockSpec((1, B), lambda i: (0, 0)),
        ],
        out_specs=pl.BlockSpec((k, D, B), lambda i: (i, 0, 0)),
        out_shape=jax.ShapeDtypeStruct((_HW, D, B), jnp.float32),
        scratch_shapes=[pltpu.VMEM((D, B), jnp.float32)],
    )(wide, parity)


def kernel(labels, table):
    B = labels.shape[0]
    V, D = table.shape
    labels = labels.astype(jnp.int32)
    table2 = table.reshape(V // 2, 2 * D)
    wide = _sc_gather(labels >> 1, table2)
    parity = (labels & 1).reshape(1, B)
    out = _tc_select_broadcast(wide, parity, D)
    return out.reshape(_H, _W, D, B).transpose(3, 2, 0, 1)
